# baseline (speedup 1.0000x reference)
import jax
import jax.numpy as jnp
from jax import lax
from jax.experimental import pallas as pl
from jax.experimental.pallas import tpu as pltpu
from jax.experimental.pallas import tpu_sc as plsc

N_ROWS = 32
ROWS = 16384
COLS = 128

CH = 4096
PER_ROW = ROWS // CH
NCH = (N_ROWS - 1) * PER_ROW
NBUF = 16
W = 8
K = NBUF - W
NGRP = -(-NCH // NBUF)


def _rd(x_hbm, buf, sem, i):
    r = 1 + i // PER_ROW
    sl = pl.ds((i % PER_ROW) * CH, CH)
    return pltpu.make_async_copy(x_hbm.at[r, sl], buf, sem)


def _wr(out_hbm, buf, sem, i):
    r = 1 + i // PER_ROW
    sl = pl.ds((i % PER_ROW) * CH, CH)
    return pltpu.make_async_copy(buf, out_hbm.at[r, sl], sem)


def _tc_body(x_hbm, out_hbm, *scratch):
    bufs = scratch[:NBUF]
    rsems = scratch[NBUF:2 * NBUF]
    wsems = scratch[2 * NBUF:]

    for j in range(K):
        _rd(x_hbm, bufs[j], rsems[j], j).start()

    def body(g, carry):
        for b in range(NBUF):
            i = g * NBUF + b

            @pl.when(i < NCH)
            def _():
                _rd(x_hbm, bufs[b], rsems[b], i).wait()
                _wr(out_hbm, bufs[b], wsems[b], i).start()

            bw = (b - W) % NBUF

            @pl.when(i >= W)
            def _():
                _wr(out_hbm, bufs[bw], wsems[bw], i - W).wait()

            br = (b + K) % NBUF

            @pl.when(i + K < NCH)
            def _():
                _rd(x_hbm, bufs[br], rsems[br], i + K).start()
        return carry

    lax.fori_loop(0, NGRP, body, 0)
    for i in range(NGRP * NBUF - W, NCH):
        b = i % NBUF
        _wr(out_hbm, bufs[b], wsems[b], i).wait()


_tc_pass_through = pl.pallas_call(
    _tc_body,
    out_shape=jax.ShapeDtypeStruct((N_ROWS, ROWS, COLS), jnp.float32),
    in_specs=[pl.BlockSpec(memory_space=pltpu.MemorySpace.HBM)],
    out_specs=pl.BlockSpec(memory_space=pltpu.MemorySpace.HBM),
    scratch_shapes=(
        [pltpu.VMEM((CH, COLS), jnp.float32) for _ in range(NBUF)]
        + [pltpu.SemaphoreType.DMA for _ in range(2 * NBUF)]
    ),
)


SC_STRIPE = ROWS // 32


def _sc_slot0_body(src_hbm, out_hbm, buf):
    c = lax.axis_index("c")
    s = lax.axis_index("s")
    w = s * 2 + c
    sl = pl.ds(w * SC_STRIPE, SC_STRIPE)
    pltpu.sync_copy(src_hbm.at[sl], buf)
    pltpu.sync_copy(buf, out_hbm.at[0, sl])


_sc_slot0 = pl.kernel(
    _sc_slot0_body,
    out_type=(),
    mesh=plsc.VectorSubcoreMesh(core_axis_name="c", subcore_axis_name="s"),
    scratch_types=[pltpu.VMEM((SC_STRIPE, COLS), jnp.float32)],
)


def kernel(x, src):
    out = _tc_pass_through(x)
    ref = jax.new_ref(out)
    _sc_slot0(src, ref)
    return ref[...]
